# baseline (speedup 1.0000x reference)
import functools
import math

import jax
import jax.numpy as jnp
from jax import lax
from jax.experimental import pallas as pl
from jax.experimental.pallas import tpu as pltpu
from jax.experimental.pallas import tpu_sc as plsc

NUM_ROWS = 100000
DIM = 128
BATCH = 4096 * 200
NC, NS, LANES = 2, 16, 16
NW = NC * NS
CHUNK = 128
CHUNKS_PER_W = BATCH // (NW * CHUNK)
NBUF = 5
NGROUPS = CHUNKS_PER_W // NBUF
HALF = CHUNK // 2
SCALE = math.sqrt(DIM)

_mesh = plsc.VectorSubcoreMesh(core_axis_name="c", subcore_axis_name="s")


@functools.partial(
    pl.kernel,
    out_type=jax.ShapeDtypeStruct((BATCH, DIM), jnp.float32),
    mesh=_mesh,
    scratch_types=(
        [pltpu.VMEM((CHUNKS_PER_W, CHUNK), jnp.int32)]
        + [pltpu.VMEM((CHUNK, DIM), jnp.float32) for _ in range(NBUF)]
        + [pltpu.SemaphoreType.DMA for _ in range(2 * NBUF)]
    ),
)
def _emb_lookup(idx_hbm, table_hbm, out_hbm, idx_all, *scratch):
    rows_v = scratch[:NBUF]
    sem_g = scratch[NBUF:2 * NBUF]
    sem_o = scratch[2 * NBUF:3 * NBUF]

    wid = lax.axis_index("s") * NC + lax.axis_index("c")
    base = wid * CHUNKS_PER_W

    pltpu.sync_copy(idx_hbm.at[pl.ds(base, CHUNKS_PER_W)], idx_all)

    def gat_cp(g, b):
        return pltpu.make_async_copy(
            table_hbm.at[idx_all.at[g]], rows_v[b], sem_g[b])

    def out_half_cp(g, b, h):
        return pltpu.make_async_copy(
            rows_v[b].at[pl.ds(h * HALF, HALF)],
            out_hbm.at[pl.ds((base + g) * CHUNK + h * HALF, HALF)],
            sem_o[b])

    for b in range(NBUF):
        gat_cp(b, b).start()

    def group(t, carry):
        for b in range(NBUF):
            g = t * NBUF + b
            gat_cp(0, b).wait()

            def scale_row(r, c2):
                for c in range(DIM // LANES):
                    sl = (r, pl.ds(c * LANES, LANES))
                    rows_v[b][sl] = rows_v[b][sl] * SCALE
                return c2

            lax.fori_loop(0, HALF, scale_row, 0, unroll=4)
            out_half_cp(g, b, 0).start()
            lax.fori_loop(HALF, CHUNK, scale_row, 0, unroll=4)
            out_half_cp(g, b, 1).start()

        for b in range(NBUF):
            @pl.when(t < NGROUPS - 1)
            def _():
                out_half_cp(0, b, 0).wait()
                out_half_cp(0, b, 1).wait()
                gat_cp(t * NBUF + NBUF + b, b).start()
        return carry

    lax.fori_loop(0, NGROUPS, group, 0)

    for b in range(NBUF):
        out_half_cp(0, b, 0).wait()
        out_half_cp(0, b, 1).wait()


def kernel(input, table):
    idx = input.reshape(BATCH // CHUNK, CHUNK).astype(jnp.int32)
    out = _emb_lookup(idx, table)
    return out.reshape(4096, 200, DIM)
